# Initial kernel scaffold; baseline (speedup 1.0000x reference)
import functools

import jax
import jax.numpy as jnp
from jax import lax
from jax.experimental import pallas as pl
from jax.experimental.pallas import tpu as pltpu
from jax.experimental.pallas import tpu_sc as plsc

_N = 10000
_NPAD = 10240
_D = 128
_G = 64
_H = 256
_E = 320000
_NC = 2
_NS = 16
_NW = _NC * _NS
_CHUNK = 128
_EPT = ((_E // _NW + _CHUNK - 1) // _CHUNK) * _CHUNK
_EPAD = _EPT * _NW
_RPT = _NPAD // _NS
_NBLK = 1024
_NGRID = _NPAD // _NBLK


def _spmm_sc(h, src, dst, val):
    mesh = plsc.VectorSubcoreMesh(core_axis_name="c", subcore_axis_name="s")

    @functools.partial(
        pl.kernel,
        out_type=jax.ShapeDtypeStruct((_NC * _NPAD, _D), jnp.float32),
        mesh=mesh,
        scratch_types=[
            pltpu.VMEM((_CHUNK,), jnp.int32),
            pltpu.VMEM((_CHUNK,), jnp.int32),
            pltpu.VMEM((_CHUNK,), jnp.float32),
            pltpu.VMEM((_CHUNK, _D), jnp.float32),
            pltpu.VMEM_SHARED((_NPAD, _D), jnp.float32),
            pltpu.SemaphoreType.DMA,
        ],
    )
    def spmm(h_hbm, src_hbm, dst_hbm, val_hbm, out_hbm,
             src_v, dst_v, val_v, rows_v, acc, sem):
        c = lax.axis_index("c")
        s = lax.axis_index("s")
        wid = c * _NS + s

        @pl.loop(0, _CHUNK)
        def _zero_rows(r):
            for d in range(_D // 16):
                rows_v[r, pl.ds(d * 16, 16)] = jnp.zeros((16,), jnp.float32)

        row_base = s * _RPT

        @pl.loop(0, _RPT // _CHUNK)
        def _zero_acc(j):
            pltpu.sync_copy(rows_v, acc.at[pl.ds(row_base + j * _CHUNK, _CHUNK)])

        plsc.subcore_barrier()

        edge_base = wid * _EPT

        @pl.loop(0, _EPT // _CHUNK)
        def _edges(t):
            off = edge_base + t * _CHUNK
            pltpu.sync_copy(src_hbm.at[pl.ds(off, _CHUNK)], src_v)
            pltpu.sync_copy(dst_hbm.at[pl.ds(off, _CHUNK)], dst_v)
            pltpu.sync_copy(val_hbm.at[pl.ds(off, _CHUNK)], val_v)
            pltpu.async_copy(h_hbm.at[src_v], rows_v, sem).wait()

            @pl.loop(0, _CHUNK // 16)
            def _scale(g):
                vb = val_v[pl.ds(g * 16, 16)]
                for e in range(16):
                    sc = jnp.take(vb, jnp.full((16,), e, jnp.int32),
                                  mode="promise_in_bounds")
                    r = g * 16 + e
                    for d in range(_D // 16):
                        rows_v[r, pl.ds(d * 16, 16)] = (
                            rows_v[r, pl.ds(d * 16, 16)] * sc)

            pltpu.sync_copy(rows_v, acc.at[dst_v], add=True)

        plsc.subcore_barrier()

        @pl.loop(0, _RPT // _CHUNK)
        def _copy_out(j):
            r0 = row_base + j * _CHUNK
            pltpu.sync_copy(acc.at[pl.ds(r0, _CHUNK)],
                            out_hbm.at[pl.ds(c * _NPAD + r0, _CHUNK)])

    return spmm(h, src, dst, val)


def _dense_relu(parts, W, b):
    def body(p_ref, w_ref, b_ref, o_ref):
        ssum = p_ref[0] + p_ref[1]
        o_ref[...] = jax.nn.relu(
            jnp.dot(ssum, w_ref[...], preferred_element_type=jnp.float32)
            + b_ref[...])

    return pl.pallas_call(
        body,
        grid=(_NGRID,),
        in_specs=[
            pl.BlockSpec((2, _NBLK, _D), lambda i: (0, i, 0)),
            pl.BlockSpec((_D, _D), lambda i: (0, 0)),
            pl.BlockSpec((1, _D), lambda i: (0, 0)),
        ],
        out_specs=pl.BlockSpec((_NBLK, _D), lambda i: (i, 0)),
        out_shape=jax.ShapeDtypeStruct((_NPAD, _D), jnp.float32),
    )(parts, W, b.reshape(1, _D))


def _head(parts, W, b, batch3, W_p0, b_p0, W_p1, b_p1):
    def body(p_ref, w_ref, b_ref, bat_ref, wp0_ref, bp0_ref, wp1_ref,
             bp1_ref, o_ref, fp_ref):
        i = pl.program_id(0)
        h2 = jax.nn.relu(
            jnp.dot(p_ref[0] + p_ref[1], w_ref[...],
                    preferred_element_type=jnp.float32) + b_ref[...])
        bb = bat_ref[0, 0, :]
        oh = (bb[:, None] == lax.broadcasted_iota(jnp.int32, (_NBLK, _G), 1)
              ).astype(jnp.float32)
        part = lax.dot_general(oh, h2, (((0,), (0,)), ((), ())),
                               preferred_element_type=jnp.float32)

        @pl.when(i == 0)
        def _init():
            fp_ref[...] = part

        @pl.when(i > 0)
        def _accum():
            fp_ref[...] = fp_ref[...] + part

        @pl.when(i == _NGRID - 1)
        def _final():
            z = jax.nn.relu(
                jnp.dot(fp_ref[...], wp0_ref[...],
                        preferred_element_type=jnp.float32) + bp0_ref[...])
            o_ref[...] = (jnp.dot(z, wp1_ref[...],
                                  preferred_element_type=jnp.float32)
                          + bp1_ref[...])

    return pl.pallas_call(
        body,
        grid=(_NGRID,),
        in_specs=[
            pl.BlockSpec((2, _NBLK, _D), lambda i: (0, i, 0)),
            pl.BlockSpec((_D, _D), lambda i: (0, 0)),
            pl.BlockSpec((1, _D), lambda i: (0, 0)),
            pl.BlockSpec((1, 1, _NBLK), lambda i: (i, 0, 0)),
            pl.BlockSpec((_D, _H), lambda i: (0, 0)),
            pl.BlockSpec((1, _H), lambda i: (0, 0)),
            pl.BlockSpec((_H, 1), lambda i: (0, 0)),
            pl.BlockSpec((1, 1), lambda i: (0, 0)),
        ],
        out_specs=pl.BlockSpec((_G, 1), lambda i: (0, 0)),
        out_shape=jax.ShapeDtypeStruct((_G, 1), jnp.float32),
        scratch_shapes=[pltpu.VMEM((_G, _D), jnp.float32)],
    )(parts, W, b.reshape(1, _D), batch3, W_p0, b_p0.reshape(1, _H),
      W_p1, b_p1.reshape(1, 1))


def kernel(node_attr, adj_index, adj_value, batch,
           W_g0, b_g0, W_g1, b_g1, W_p0, b_p0, W_p1, b_p1):
    dst = adj_index[0]
    src = adj_index[1]
    pad_e = _EPAD - _E
    src_p = jnp.concatenate([src.astype(jnp.int32),
                             jnp.zeros((pad_e,), jnp.int32)])
    dst_p = jnp.concatenate([dst.astype(jnp.int32),
                             jnp.zeros((pad_e,), jnp.int32)])
    val_p = jnp.concatenate([adj_value, jnp.zeros((pad_e,), jnp.float32)])
    batch3 = jnp.concatenate(
        [batch.astype(jnp.int32), jnp.full((_NPAD - _N,), _G, jnp.int32)]
    ).reshape(_NGRID, 1, _NBLK)

    parts0 = _spmm_sc(node_attr, src_p, dst_p, val_p).reshape(_NC, _NPAD, _D)
    h1 = _dense_relu(parts0, W_g0, b_g0)
    parts1 = _spmm_sc(h1, src_p, dst_p, val_p).reshape(_NC, _NPAD, _D)
    return _head(parts1, W_g1, b_g1, batch3, W_p0, b_p0, W_p1, b_p1)


# speedup vs baseline: 3.8442x; 3.8442x over previous
import functools

import jax
import jax.numpy as jnp
from jax import lax
from jax.experimental import pallas as pl
from jax.experimental.pallas import tpu as pltpu
from jax.experimental.pallas import tpu_sc as plsc

_N = 10000
_NPAD = 10240
_D = 128
_G = 64
_H = 256
_E = 320000
_NC = 2
_NS = 16
_NW = _NC * _NS
_CHUNK = 128
_EPT = ((_E // _NW + _CHUNK - 1) // _CHUNK) * _CHUNK
_EPAD = _EPT * _NW
_RPT = _NPAD // _NS
_NBLK = 1024
_NGRID = _NPAD // _NBLK

_GATHER_DN = lax.GatherDimensionNumbers(
    offset_dims=(), collapsed_slice_dims=(0,), start_index_map=(0,))


def _bcast_lane(vec16, lane):
    idx = jnp.full((16, 1), lane, jnp.int32)
    return lax.gather(vec16, idx, _GATHER_DN, slice_sizes=(1,),
                      mode=lax.GatherScatterMode.PROMISE_IN_BOUNDS)


def _spmm_sc(h, src, dst, val):
    mesh = plsc.VectorSubcoreMesh(core_axis_name="c", subcore_axis_name="s")

    @functools.partial(
        pl.kernel,
        out_type=jax.ShapeDtypeStruct((_NC * _NPAD, _D), jnp.float32),
        mesh=mesh,
        scratch_types=[
            pltpu.VMEM((_CHUNK,), jnp.int32),
            pltpu.VMEM((_CHUNK,), jnp.int32),
            pltpu.VMEM((_CHUNK,), jnp.float32),
            pltpu.VMEM((_CHUNK, _D), jnp.float32),
            pltpu.VMEM_SHARED((_NPAD, _D), jnp.float32),
            pltpu.SemaphoreType.DMA,
        ],
    )
    def spmm(h_hbm, src_hbm, dst_hbm, val_hbm, out_hbm,
             src_v, dst_v, val_v, rows_v, acc, sem):
        c = lax.axis_index("c")
        s = lax.axis_index("s")
        wid = c * _NS + s

        @pl.loop(0, _CHUNK)
        def _zero_rows(r):
            for d in range(_D // 16):
                rows_v[r, pl.ds(d * 16, 16)] = jnp.zeros((16,), jnp.float32)

        row_base = s * _RPT

        @pl.loop(0, _RPT // _CHUNK)
        def _zero_acc(j):
            pltpu.sync_copy(rows_v, acc.at[pl.ds(row_base + j * _CHUNK, _CHUNK)])

        plsc.subcore_barrier()

        edge_base = wid * _EPT

        @pl.loop(0, _EPT // _CHUNK)
        def _edges(t):
            off = edge_base + t * _CHUNK
            pltpu.sync_copy(src_hbm.at[pl.ds(off, _CHUNK)], src_v)
            pltpu.sync_copy(dst_hbm.at[pl.ds(off, _CHUNK)], dst_v)
            pltpu.sync_copy(val_hbm.at[pl.ds(off, _CHUNK)], val_v)
            pltpu.async_copy(h_hbm.at[src_v], rows_v, sem).wait()

            @pl.loop(0, _CHUNK // 16)
            def _scale(g):
                vb = val_v[pl.ds(g * 16, 16)]
                for e in range(16):
                    sc = _bcast_lane(vb, e)
                    r = g * 16 + e
                    for d in range(_D // 16):
                        rows_v[r, pl.ds(d * 16, 16)] = (
                            rows_v[r, pl.ds(d * 16, 16)] * sc)

            pltpu.sync_copy(rows_v, acc.at[dst_v], add=True)

        plsc.subcore_barrier()

        @pl.loop(0, _RPT // _CHUNK)
        def _copy_out(j):
            r0 = row_base + j * _CHUNK
            pltpu.sync_copy(acc.at[pl.ds(r0, _CHUNK)],
                            out_hbm.at[pl.ds(c * _NPAD + r0, _CHUNK)])

    return spmm(h, src, dst, val)


def _dense_relu(parts, W, b):
    def body(p_ref, w_ref, b_ref, o_ref):
        ssum = p_ref[0] + p_ref[1]
        o_ref[...] = jax.nn.relu(
            jnp.dot(ssum, w_ref[...], preferred_element_type=jnp.float32)
            + b_ref[...])

    return pl.pallas_call(
        body,
        grid=(_NGRID,),
        in_specs=[
            pl.BlockSpec((2, _NBLK, _D), lambda i: (0, i, 0)),
            pl.BlockSpec((_D, _D), lambda i: (0, 0)),
            pl.BlockSpec((1, _D), lambda i: (0, 0)),
        ],
        out_specs=pl.BlockSpec((_NBLK, _D), lambda i: (i, 0)),
        out_shape=jax.ShapeDtypeStruct((_NPAD, _D), jnp.float32),
    )(parts, W, b.reshape(1, _D))


def _head(parts, W, b, batch3, W_p0, b_p0, W_p1, b_p1):
    def body(p_ref, w_ref, b_ref, bat_ref, wp0_ref, bp0_ref, wp1_ref,
             bp1_ref, o_ref, fp_ref):
        i = pl.program_id(0)
        h2 = jax.nn.relu(
            jnp.dot(p_ref[0] + p_ref[1], w_ref[...],
                    preferred_element_type=jnp.float32) + b_ref[...])
        bb = bat_ref[0, 0, :]
        oh = (bb[:, None] == lax.broadcasted_iota(jnp.int32, (_NBLK, _G), 1)
              ).astype(jnp.float32)
        part = lax.dot_general(oh, h2, (((0,), (0,)), ((), ())),
                               preferred_element_type=jnp.float32)

        @pl.when(i == 0)
        def _init():
            fp_ref[...] = part

        @pl.when(i > 0)
        def _accum():
            fp_ref[...] = fp_ref[...] + part

        @pl.when(i == _NGRID - 1)
        def _final():
            z = jax.nn.relu(
                jnp.dot(fp_ref[...], wp0_ref[...],
                        preferred_element_type=jnp.float32) + bp0_ref[...])
            o_ref[...] = (jnp.dot(z, wp1_ref[...],
                                  preferred_element_type=jnp.float32)
                          + bp1_ref[...])

    return pl.pallas_call(
        body,
        grid=(_NGRID,),
        in_specs=[
            pl.BlockSpec((2, _NBLK, _D), lambda i: (0, i, 0)),
            pl.BlockSpec((_D, _D), lambda i: (0, 0)),
            pl.BlockSpec((1, _D), lambda i: (0, 0)),
            pl.BlockSpec((1, 1, _NBLK), lambda i: (i, 0, 0)),
            pl.BlockSpec((_D, _H), lambda i: (0, 0)),
            pl.BlockSpec((1, _H), lambda i: (0, 0)),
            pl.BlockSpec((_H, 1), lambda i: (0, 0)),
            pl.BlockSpec((1, 1), lambda i: (0, 0)),
        ],
        out_specs=pl.BlockSpec((_G, 1), lambda i: (0, 0)),
        out_shape=jax.ShapeDtypeStruct((_G, 1), jnp.float32),
        scratch_shapes=[pltpu.VMEM((_G, _D), jnp.float32)],
    )(parts, W, b.reshape(1, _D), batch3, W_p0, b_p0.reshape(1, _H),
      W_p1, b_p1.reshape(1, 1))


def kernel(node_attr, adj_index, adj_value, batch,
           W_g0, b_g0, W_g1, b_g1, W_p0, b_p0, W_p1, b_p1):
    dst = adj_index[0]
    src = adj_index[1]
    pad_e = _EPAD - _E
    src_p = jnp.concatenate([src.astype(jnp.int32),
                             jnp.zeros((pad_e,), jnp.int32)])
    dst_p = jnp.concatenate([dst.astype(jnp.int32),
                             jnp.zeros((pad_e,), jnp.int32)])
    val_p = jnp.concatenate([adj_value, jnp.zeros((pad_e,), jnp.float32)])
    batch3 = jnp.concatenate(
        [batch.astype(jnp.int32), jnp.full((_NPAD - _N,), _G, jnp.int32)]
    ).reshape(_NGRID, 1, _NBLK)

    parts0 = _spmm_sc(node_attr, src_p, dst_p, val_p).reshape(_NC, _NPAD, _D)
    h1 = _dense_relu(parts0, W_g0, b_g0)
    parts1 = _spmm_sc(h1, src_p, dst_p, val_p).reshape(_NC, _NPAD, _D)
    return _head(parts1, W_g1, b_g1, batch3, W_p0, b_p0, W_p1, b_p1)


# speedup vs baseline: 12.0462x; 3.1336x over previous
import dataclasses
import functools

import jax
import jax.numpy as jnp
from jax import lax
from jax.experimental import pallas as pl
from jax.experimental.pallas import tpu as pltpu
from jax.experimental.pallas import tpu_sc as plsc

_N = 10000
_NPAD = 10240
_D = 128
_G = 64
_H = 256
_E = 320000
_NC = 2
_NS = 16
_NW = _NC * _NS
_CHUNK = 128
_NCH = 84
_NITER = _NCH // 12
_EPT = _NCH * _CHUNK
_EPAD = _EPT * _NW
_NACC = 10000
_RPT = 624
_NBLK = 1024
_NGRID = _NPAD // _NBLK

_GATHER_DN = lax.GatherDimensionNumbers(
    offset_dims=(), collapsed_slice_dims=(0,), start_index_map=(0,))


def _bcast_lane(vec16, lane):
    idx = jnp.full((16, 1), lane, jnp.int32)
    return lax.gather(vec16, idx, _GATHER_DN, slice_sizes=(1,),
                      mode=lax.GatherScatterMode.PROMISE_IN_BOUNDS)


def _spmm_sc(h, src, dst, val):
    mesh = plsc.VectorSubcoreMesh(core_axis_name="c", subcore_axis_name="s")
    cp = pltpu.CompilerParams()
    if "needs_layout_passes" in pltpu.CompilerParams.__dataclass_fields__:
        cp = dataclasses.replace(cp, needs_layout_passes=False)

    ring_types = ([pltpu.VMEM((_CHUNK,), jnp.int32)] * 8
                  + [pltpu.VMEM((_CHUNK,), jnp.float32)] * 4)

    @functools.partial(
        pl.kernel,
        out_type=jax.ShapeDtypeStruct((_NC * _NPAD, _D), jnp.float32),
        mesh=mesh,
        compiler_params=cp,
        scratch_types=ring_types + [
            pltpu.VMEM((_CHUNK, _D), jnp.float32),
            pltpu.VMEM((_CHUNK, _D), jnp.float32),
            pltpu.VMEM((_CHUNK, _D), jnp.float32),
            pltpu.VMEM_SHARED((_NACC, _D), jnp.float32),
        ] + [pltpu.SemaphoreType.DMA] * 10,
    )
    def spmm(h_hbm, src_hbm, dst_hbm, val_hbm, out_hbm,
             sr0, sr1, sr2, sr3, dr0, dr1, dr2, dr3, vr0, vr1, vr2, vr3,
             rows0, rows1, rows2, acc,
             gs0, gs1, gs2, ss0, ss1, ss2, is0, is1, is2, is3):
        srcr = (sr0, sr1, sr2, sr3)
        dstr = (dr0, dr1, dr2, dr3)
        valr = (vr0, vr1, vr2, vr3)
        bufs = (rows0, rows1, rows2)
        gsems = (gs0, gs1, gs2)
        ssems = (ss0, ss1, ss2)
        isems = (is0, is1, is2, is3)
        c = lax.axis_index("c")
        s = lax.axis_index("s")
        wid = c * _NS + s
        ebase = wid * _EPT

        @pl.loop(0, _CHUNK)
        def _zero_rows(r):
            for d in range(_D // 16):
                rows0[r, pl.ds(d * 16, 16)] = jnp.zeros((16,), jnp.float32)

        row_base = s * _RPT

        @pl.loop(0, _RPT // 104)
        def _zero_acc(j):
            pltpu.sync_copy(rows0.at[pl.ds(0, 104)],
                            acc.at[pl.ds(row_base + j * 104, 104)])

        @pl.when(s == _NS - 1)
        def _zero_acc_tail():
            pltpu.sync_copy(rows0.at[pl.ds(0, 16)],
                            acc.at[pl.ds(_NS * _RPT, 16)])

        @pl.when(s == 0)
        def _zero_out_pad():
            pltpu.sync_copy(
                rows0.at[pl.ds(0, 120)],
                out_hbm.at[pl.ds(c * _NPAD + _NACC, 120)])
            pltpu.sync_copy(
                rows0.at[pl.ds(0, 120)],
                out_hbm.at[pl.ds(c * _NPAD + _NACC + 120, 120)])

        plsc.subcore_barrier()

        def _scale_rows(buf, vref):
            @pl.loop(0, _CHUNK // 16)
            def _scale(g):
                vb = vref[pl.ds(g * 16, 16)]
                for e in range(16):
                    sc = _bcast_lane(vb, e)
                    r = g * 16 + e
                    for d in range(_D // 16):
                        buf[r, pl.ds(d * 16, 16)] = (
                            buf[r, pl.ds(d * 16, 16)] * sc)

        def _stage_idx(t, sl, sync=False):
            off = ebase + t * _CHUNK
            if sync:
                pltpu.sync_copy(src_hbm.at[pl.ds(off, _CHUNK)], srcr[sl])
                pltpu.sync_copy(dst_hbm.at[pl.ds(off, _CHUNK)], dstr[sl])
                pltpu.sync_copy(val_hbm.at[pl.ds(off, _CHUNK)], valr[sl])
            else:
                pltpu.async_copy(src_hbm.at[pl.ds(off, _CHUNK)], srcr[sl],
                                 isems[sl])
                pltpu.async_copy(dst_hbm.at[pl.ds(off, _CHUNK)], dstr[sl],
                                 isems[sl])
                pltpu.async_copy(val_hbm.at[pl.ds(off, _CHUNK)], valr[sl],
                                 isems[sl])

        def _wait_idx_sem(t, sl):
            off = ebase + t * _CHUNK
            pltpu.make_async_copy(src_hbm.at[pl.ds(off, _CHUNK)], srcr[sl],
                                  isems[sl]).wait()
            pltpu.make_async_copy(dst_hbm.at[pl.ds(off, _CHUNK)], dstr[sl],
                                  isems[sl]).wait()
            pltpu.make_async_copy(val_hbm.at[pl.ds(off, _CHUNK)], valr[sl],
                                  isems[sl]).wait()

        _stage_idx(0, 0, sync=True)
        _stage_idx(1, 1, sync=True)
        pltpu.async_copy(h_hbm.at[srcr[0]], rows0, gs0)

        @pl.loop(0, _NITER)
        def _body(i):
            for j in range(12):
                t = i * 12 + j
                k = j % 3
                kn = (k + 1) % 3
                sl = j % 4
                sln = (j + 1) % 4
                slp = (j + 2) % 4

                def _wait_sca(kn=kn, slp=slp):
                    pltpu.make_async_copy(
                        bufs[kn], acc.at[dstr[slp]], ssems[kn]).wait()
                if j >= 2:
                    _wait_sca()
                else:
                    pl.when(i > 0)(_wait_sca)

                def _stage(t=t, slp=slp):
                    _stage_idx(t + 2, slp)
                if j < 10:
                    _stage()
                else:
                    pl.when(i < _NITER - 1)(_stage)

                def _wait_idx(t=t, sln=sln):
                    _wait_idx_sem(t + 1, sln)
                if j == 0:
                    pl.when(i > 0)(_wait_idx)
                elif j == 11:
                    pl.when(i < _NITER - 1)(_wait_idx)
                else:
                    _wait_idx()

                def _gather(kn=kn, sln=sln):
                    pltpu.async_copy(h_hbm.at[srcr[sln]], bufs[kn],
                                     gsems[kn])
                if j == 11:
                    pl.when(i < _NITER - 1)(_gather)
                else:
                    _gather()

                pltpu.make_async_copy(
                    h_hbm.at[srcr[sl]], bufs[k], gsems[k]).wait()
                _scale_rows(bufs[k], valr[sl])
                pltpu.async_copy(bufs[k], acc.at[dstr[sl]], ssems[k],
                                 add=True)

        pltpu.make_async_copy(
            bufs[(_NCH - 2) % 3], acc.at[dstr[(_NCH - 2) % 4]],
            ssems[(_NCH - 2) % 3]).wait()
        pltpu.make_async_copy(
            bufs[(_NCH - 1) % 3], acc.at[dstr[(_NCH - 1) % 4]],
            ssems[(_NCH - 1) % 3]).wait()

        plsc.subcore_barrier()

        @pl.loop(0, _RPT // 208)
        def _copy_out(j):
            r0 = row_base + j * 208
            pltpu.sync_copy(acc.at[pl.ds(r0, 208)],
                            out_hbm.at[pl.ds(c * _NPAD + r0, 208)])

        @pl.when(s == _NS - 1)
        def _copy_out_tail():
            pltpu.sync_copy(acc.at[pl.ds(_NS * _RPT, 16)],
                            out_hbm.at[pl.ds(c * _NPAD + _NS * _RPT, 16)])

    return spmm(h, src, dst, val)


def _dense_relu(parts, W, b):
    def body(p_ref, w_ref, b_ref, o_ref):
        ssum = p_ref[0] + p_ref[1]
        o_ref[...] = jax.nn.relu(
            jnp.dot(ssum, w_ref[...], preferred_element_type=jnp.float32)
            + b_ref[...])

    return pl.pallas_call(
        body,
        grid=(_NGRID,),
        in_specs=[
            pl.BlockSpec((2, _NBLK, _D), lambda i: (0, i, 0)),
            pl.BlockSpec((_D, _D), lambda i: (0, 0)),
            pl.BlockSpec((1, _D), lambda i: (0, 0)),
        ],
        out_specs=pl.BlockSpec((_NBLK, _D), lambda i: (i, 0)),
        out_shape=jax.ShapeDtypeStruct((_NPAD, _D), jnp.float32),
    )(parts, W, b.reshape(1, _D))


def _head(parts, W, b, batch3, W_p0, b_p0, W_p1, b_p1):
    def body(p_ref, w_ref, b_ref, bat_ref, wp0_ref, bp0_ref, wp1_ref,
             bp1_ref, o_ref, fp_ref):
        i = pl.program_id(0)
        h2 = jax.nn.relu(
            jnp.dot(p_ref[0] + p_ref[1], w_ref[...],
                    preferred_element_type=jnp.float32) + b_ref[...])
        bb = bat_ref[0, 0, :]
        oh = (bb[:, None] == lax.broadcasted_iota(jnp.int32, (_NBLK, _G), 1)
              ).astype(jnp.float32)
        part = lax.dot_general(oh, h2, (((0,), (0,)), ((), ())),
                               preferred_element_type=jnp.float32)

        @pl.when(i == 0)
        def _init():
            fp_ref[...] = part

        @pl.when(i > 0)
        def _accum():
            fp_ref[...] = fp_ref[...] + part

        @pl.when(i == _NGRID - 1)
        def _final():
            z = jax.nn.relu(
                jnp.dot(fp_ref[...], wp0_ref[...],
                        preferred_element_type=jnp.float32) + bp0_ref[...])
            o_ref[...] = (jnp.dot(z, wp1_ref[...],
                                  preferred_element_type=jnp.float32)
                          + bp1_ref[...])

    return pl.pallas_call(
        body,
        grid=(_NGRID,),
        in_specs=[
            pl.BlockSpec((2, _NBLK, _D), lambda i: (0, i, 0)),
            pl.BlockSpec((_D, _D), lambda i: (0, 0)),
            pl.BlockSpec((1, _D), lambda i: (0, 0)),
            pl.BlockSpec((1, 1, _NBLK), lambda i: (i, 0, 0)),
            pl.BlockSpec((_D, _H), lambda i: (0, 0)),
            pl.BlockSpec((1, _H), lambda i: (0, 0)),
            pl.BlockSpec((_H, 1), lambda i: (0, 0)),
            pl.BlockSpec((1, 1), lambda i: (0, 0)),
        ],
        out_specs=pl.BlockSpec((_G, 1), lambda i: (0, 0)),
        out_shape=jax.ShapeDtypeStruct((_G, 1), jnp.float32),
        scratch_shapes=[pltpu.VMEM((_G, _D), jnp.float32)],
    )(parts, W, b.reshape(1, _D), batch3, W_p0, b_p0.reshape(1, _H),
      W_p1, b_p1.reshape(1, 1))


def kernel(node_attr, adj_index, adj_value, batch,
           W_g0, b_g0, W_g1, b_g1, W_p0, b_p0, W_p1, b_p1):
    dst = adj_index[0]
    src = adj_index[1]
    pad_e = _EPAD - _E
    pad_idx = (jnp.arange(pad_e, dtype=jnp.int32) * 61) % _N
    src_p = jnp.concatenate([src.astype(jnp.int32), pad_idx])
    dst_p = jnp.concatenate([dst.astype(jnp.int32), pad_idx])
    val_p = jnp.concatenate([adj_value, jnp.zeros((pad_e,), jnp.float32)])
    batch3 = jnp.concatenate(
        [batch.astype(jnp.int32), jnp.full((_NPAD - _N,), _G, jnp.int32)]
    ).reshape(_NGRID, 1, _NBLK)

    parts0 = _spmm_sc(node_attr, src_p, dst_p, val_p).reshape(_NC, _NPAD, _D)
    h1 = _dense_relu(parts0, W_g0, b_g0)
    parts1 = _spmm_sc(h1, src_p, dst_p, val_p).reshape(_NC, _NPAD, _D)
    return _head(parts1, W_g1, b_g1, batch3, W_p0, b_p0, W_p1, b_p1)


# speedup vs baseline: 12.1596x; 1.0094x over previous
import dataclasses
import functools

import jax
import jax.numpy as jnp
from jax import lax
from jax.experimental import pallas as pl
from jax.experimental.pallas import tpu as pltpu
from jax.experimental.pallas import tpu_sc as plsc

_N = 10000
_NPAD = 10240
_D = 128
_G = 64
_H = 256
_E = 320000
_NC = 2
_NS = 16
_NW = _NC * _NS
_CHUNK = 128
_NCH = 84
_NITER = _NCH // 12
_EPT = _NCH * _CHUNK
_EPAD = _EPT * _NW
_NACC = 10000
_RPT = 624
_NBLK = 1024
_NGRID = _NPAD // _NBLK

_GATHER_DN = lax.GatherDimensionNumbers(
    offset_dims=(), collapsed_slice_dims=(0,), start_index_map=(0,))


def _bcast_lane(vec16, lane):
    idx = jnp.full((16, 1), lane, jnp.int32)
    return lax.gather(vec16, idx, _GATHER_DN, slice_sizes=(1,),
                      mode=lax.GatherScatterMode.PROMISE_IN_BOUNDS)


def _spmm_sc(h, src, dst, val):
    mesh = plsc.VectorSubcoreMesh(core_axis_name="c", subcore_axis_name="s")
    cp = pltpu.CompilerParams()
    if "needs_layout_passes" in pltpu.CompilerParams.__dataclass_fields__:
        cp = dataclasses.replace(cp, needs_layout_passes=False)

    ring_types = ([pltpu.VMEM((_CHUNK,), jnp.int32)] * 8
                  + [pltpu.VMEM((_CHUNK,), jnp.float32)] * 4)

    @functools.partial(
        pl.kernel,
        out_type=jax.ShapeDtypeStruct((_NC * _NPAD, _D), jnp.float32),
        mesh=mesh,
        compiler_params=cp,
        scratch_types=ring_types + [
            pltpu.VMEM((_CHUNK, _D), jnp.float32),
            pltpu.VMEM((_CHUNK, _D), jnp.float32),
            pltpu.VMEM((_CHUNK, _D), jnp.float32),
            pltpu.VMEM_SHARED((_NACC, _D), jnp.float32),
        ] + [pltpu.SemaphoreType.DMA] * 10,
    )
    def spmm(h_hbm, src_hbm, dst_hbm, val_hbm, out_hbm,
             sr0, sr1, sr2, sr3, dr0, dr1, dr2, dr3, vr0, vr1, vr2, vr3,
             rows0, rows1, rows2, acc,
             gs0, gs1, gs2, ss0, ss1, ss2, is0, is1, is2, is3):
        srcr = (sr0, sr1, sr2, sr3)
        dstr = (dr0, dr1, dr2, dr3)
        valr = (vr0, vr1, vr2, vr3)
        bufs = (rows0, rows1, rows2)
        gsems = (gs0, gs1, gs2)
        ssems = (ss0, ss1, ss2)
        isems = (is0, is1, is2, is3)
        c = lax.axis_index("c")
        s = lax.axis_index("s")
        wid = c * _NS + s
        ebase = wid * _EPT

        def _stage_idx(t, sl, sync=False):
            off = ebase + t * _CHUNK
            if sync:
                pltpu.sync_copy(src_hbm.at[pl.ds(off, _CHUNK)], srcr[sl])
                pltpu.sync_copy(dst_hbm.at[pl.ds(off, _CHUNK)], dstr[sl])
                pltpu.sync_copy(val_hbm.at[pl.ds(off, _CHUNK)], valr[sl])
            else:
                pltpu.async_copy(src_hbm.at[pl.ds(off, _CHUNK)], srcr[sl],
                                 isems[sl])
                pltpu.async_copy(dst_hbm.at[pl.ds(off, _CHUNK)], dstr[sl],
                                 isems[sl])
                pltpu.async_copy(val_hbm.at[pl.ds(off, _CHUNK)], valr[sl],
                                 isems[sl])

        @pl.loop(0, _CHUNK)
        def _zero_rows(r):
            for d in range(_D // 16):
                rows2[r, pl.ds(d * 16, 16)] = jnp.zeros((16,), jnp.float32)

        _stage_idx(0, 0, sync=True)
        pltpu.async_copy(h_hbm.at[srcr[0]], rows0, gs0)
        _stage_idx(1, 1, sync=True)

        row_base = s * _RPT

        @pl.loop(0, _RPT // 104)
        def _zero_acc(j):
            pltpu.sync_copy(rows2.at[pl.ds(0, 104)],
                            acc.at[pl.ds(row_base + j * 104, 104)])

        @pl.when(s == _NS - 1)
        def _zero_acc_tail():
            pltpu.sync_copy(rows2.at[pl.ds(0, 16)],
                            acc.at[pl.ds(_NS * _RPT, 16)])

        @pl.when(s == 0)
        def _zero_out_pad():
            pltpu.sync_copy(
                rows2.at[pl.ds(0, 120)],
                out_hbm.at[pl.ds(c * _NPAD + _NACC, 120)])
            pltpu.sync_copy(
                rows2.at[pl.ds(0, 120)],
                out_hbm.at[pl.ds(c * _NPAD + _NACC + 120, 120)])

        plsc.subcore_barrier()

        def _scale_rows(buf, vref):
            @pl.loop(0, _CHUNK // 16)
            def _scale(g):
                vb = vref[pl.ds(g * 16, 16)]
                for e in range(16):
                    sc = _bcast_lane(vb, e)
                    r = g * 16 + e
                    for d in range(_D // 16):
                        buf[r, pl.ds(d * 16, 16)] = (
                            buf[r, pl.ds(d * 16, 16)] * sc)

        def _wait_idx_sem(t, sl):
            off = ebase + t * _CHUNK
            pltpu.make_async_copy(src_hbm.at[pl.ds(off, _CHUNK)], srcr[sl],
                                  isems[sl]).wait()
            pltpu.make_async_copy(dst_hbm.at[pl.ds(off, _CHUNK)], dstr[sl],
                                  isems[sl]).wait()
            pltpu.make_async_copy(val_hbm.at[pl.ds(off, _CHUNK)], valr[sl],
                                  isems[sl]).wait()

        @pl.loop(0, _NITER)
        def _body(i):
            for j in range(12):
                t = i * 12 + j
                k = j % 3
                kn = (k + 1) % 3
                sl = j % 4
                sln = (j + 1) % 4
                slp = (j + 2) % 4

                def _wait_sca(kn=kn, slp=slp):
                    pltpu.make_async_copy(
                        bufs[kn], acc.at[dstr[slp]], ssems[kn]).wait()
                if j >= 2:
                    _wait_sca()
                else:
                    pl.when(i > 0)(_wait_sca)

                def _stage(t=t, slp=slp):
                    _stage_idx(t + 2, slp)
                if j < 10:
                    _stage()
                else:
                    pl.when(i < _NITER - 1)(_stage)

                def _wait_idx(t=t, sln=sln):
                    _wait_idx_sem(t + 1, sln)
                if j == 0:
                    pl.when(i > 0)(_wait_idx)
                elif j == 11:
                    pl.when(i < _NITER - 1)(_wait_idx)
                else:
                    _wait_idx()

                def _gather(kn=kn, sln=sln):
                    pltpu.async_copy(h_hbm.at[srcr[sln]], bufs[kn],
                                     gsems[kn])
                if j == 11:
                    pl.when(i < _NITER - 1)(_gather)
                else:
                    _gather()

                pltpu.make_async_copy(
                    h_hbm.at[srcr[sl]], bufs[k], gsems[k]).wait()
                _scale_rows(bufs[k], valr[sl])
                pltpu.async_copy(bufs[k], acc.at[dstr[sl]], ssems[k],
                                 add=True)

        pltpu.make_async_copy(
            bufs[(_NCH - 2) % 3], acc.at[dstr[(_NCH - 2) % 4]],
            ssems[(_NCH - 2) % 3]).wait()
        pltpu.make_async_copy(
            bufs[(_NCH - 1) % 3], acc.at[dstr[(_NCH - 1) % 4]],
            ssems[(_NCH - 1) % 3]).wait()

        plsc.subcore_barrier()

        @pl.loop(0, _RPT // 208)
        def _copy_out(j):
            r0 = row_base + j * 208
            pltpu.sync_copy(acc.at[pl.ds(r0, 208)],
                            out_hbm.at[pl.ds(c * _NPAD + r0, 208)])

        @pl.when(s == _NS - 1)
        def _copy_out_tail():
            pltpu.sync_copy(acc.at[pl.ds(_NS * _RPT, 16)],
                            out_hbm.at[pl.ds(c * _NPAD + _NS * _RPT, 16)])

    return spmm(h, src, dst, val)


def _dense_relu(parts, W, b):
    def body(p_ref, w_ref, b_ref, o_ref):
        ssum = p_ref[0] + p_ref[1]
        o_ref[...] = jax.nn.relu(
            jnp.dot(ssum, w_ref[...], preferred_element_type=jnp.float32)
            + b_ref[...])

    return pl.pallas_call(
        body,
        grid=(_NGRID,),
        in_specs=[
            pl.BlockSpec((2, _NBLK, _D), lambda i: (0, i, 0)),
            pl.BlockSpec((_D, _D), lambda i: (0, 0)),
            pl.BlockSpec((1, _D), lambda i: (0, 0)),
        ],
        out_specs=pl.BlockSpec((_NBLK, _D), lambda i: (i, 0)),
        out_shape=jax.ShapeDtypeStruct((_NPAD, _D), jnp.float32),
    )(parts, W, b.reshape(1, _D))


def _head(parts, W, b, batch3, W_p0, b_p0, W_p1, b_p1):
    def body(p_ref, w_ref, b_ref, bat_ref, wp0_ref, bp0_ref, wp1_ref,
             bp1_ref, o_ref, fp_ref):
        i = pl.program_id(0)
        h2 = jax.nn.relu(
            jnp.dot(p_ref[0] + p_ref[1], w_ref[...],
                    preferred_element_type=jnp.float32) + b_ref[...])
        bb = bat_ref[0, 0, :]
        oh = (bb[:, None] == lax.broadcasted_iota(jnp.int32, (_NBLK, _G), 1)
              ).astype(jnp.float32)
        part = lax.dot_general(oh, h2, (((0,), (0,)), ((), ())),
                               preferred_element_type=jnp.float32)

        @pl.when(i == 0)
        def _init():
            fp_ref[...] = part

        @pl.when(i > 0)
        def _accum():
            fp_ref[...] = fp_ref[...] + part

        @pl.when(i == _NGRID - 1)
        def _final():
            z = jax.nn.relu(
                jnp.dot(fp_ref[...], wp0_ref[...],
                        preferred_element_type=jnp.float32) + bp0_ref[...])
            o_ref[...] = (jnp.dot(z, wp1_ref[...],
                                  preferred_element_type=jnp.float32)
                          + bp1_ref[...])

    return pl.pallas_call(
        body,
        grid=(_NGRID,),
        in_specs=[
            pl.BlockSpec((2, _NBLK, _D), lambda i: (0, i, 0)),
            pl.BlockSpec((_D, _D), lambda i: (0, 0)),
            pl.BlockSpec((1, _D), lambda i: (0, 0)),
            pl.BlockSpec((1, 1, _NBLK), lambda i: (i, 0, 0)),
            pl.BlockSpec((_D, _H), lambda i: (0, 0)),
            pl.BlockSpec((1, _H), lambda i: (0, 0)),
            pl.BlockSpec((_H, 1), lambda i: (0, 0)),
            pl.BlockSpec((1, 1), lambda i: (0, 0)),
        ],
        out_specs=pl.BlockSpec((_G, 1), lambda i: (0, 0)),
        out_shape=jax.ShapeDtypeStruct((_G, 1), jnp.float32),
        scratch_shapes=[pltpu.VMEM((_G, _D), jnp.float32)],
    )(parts, W, b.reshape(1, _D), batch3, W_p0, b_p0.reshape(1, _H),
      W_p1, b_p1.reshape(1, 1))


def kernel(node_attr, adj_index, adj_value, batch,
           W_g0, b_g0, W_g1, b_g1, W_p0, b_p0, W_p1, b_p1):
    dst = adj_index[0]
    src = adj_index[1]
    pad_e = _EPAD - _E
    pad_idx = (jnp.arange(pad_e, dtype=jnp.int32) * 61) % _N
    src_p = jnp.concatenate([src.astype(jnp.int32), pad_idx])
    dst_p = jnp.concatenate([dst.astype(jnp.int32), pad_idx])
    val_p = jnp.concatenate([adj_value, jnp.zeros((pad_e,), jnp.float32)])
    batch3 = jnp.concatenate(
        [batch.astype(jnp.int32), jnp.full((_NPAD - _N,), _G, jnp.int32)]
    ).reshape(_NGRID, 1, _NBLK)

    parts0 = _spmm_sc(node_attr, src_p, dst_p, val_p).reshape(_NC, _NPAD, _D)
    h1 = _dense_relu(parts0, W_g0, b_g0)
    parts1 = _spmm_sc(h1, src_p, dst_p, val_p).reshape(_NC, _NPAD, _D)
    return _head(parts1, W_g1, b_g1, batch3, W_p0, b_p0, W_p1, b_p1)


# speedup vs baseline: 12.8957x; 1.0605x over previous
import dataclasses
import functools

import jax
import jax.numpy as jnp
from jax import lax
from jax.experimental import pallas as pl
from jax.experimental.pallas import tpu as pltpu
from jax.experimental.pallas import tpu_sc as plsc

_N = 10000
_NPAD = 10240
_D = 128
_G = 64
_H = 256
_E = 320000
_NC = 2
_NS = 16
_NW = _NC * _NS
_CHUNK = 48
_NCH = 210
_NITER = _NCH // 6
_EPT = _NCH * _CHUNK
_EPAD = _EPT * _NW
_NACC = 10000
_RPT = 624
_NBLK = 1024
_NGRID = _NPAD // _NBLK

_GATHER_DN = lax.GatherDimensionNumbers(
    offset_dims=(), collapsed_slice_dims=(0,), start_index_map=(0,))


def _bcast_lane(vec16, lane):
    idx = jnp.full((16, 1), lane, jnp.int32)
    return lax.gather(vec16, idx, _GATHER_DN, slice_sizes=(1,),
                      mode=lax.GatherScatterMode.PROMISE_IN_BOUNDS)


def _spmm_sc(h, src, dst, val):
    mesh = plsc.VectorSubcoreMesh(core_axis_name="c", subcore_axis_name="s")
    cp = pltpu.CompilerParams()
    if "needs_layout_passes" in pltpu.CompilerParams.__dataclass_fields__:
        cp = dataclasses.replace(cp, needs_layout_passes=False)

    ring_types = ([pltpu.VMEM((_CHUNK,), jnp.int32)] * 12
                  + [pltpu.VMEM((_CHUNK,), jnp.float32)] * 6)

    @functools.partial(
        pl.kernel,
        out_type=jax.ShapeDtypeStruct((_NC * _NPAD, _D), jnp.float32),
        mesh=mesh,
        compiler_params=cp,
        scratch_types=ring_types
        + [pltpu.VMEM((_CHUNK, _D), jnp.float32)] * 6
        + [pltpu.VMEM_SHARED((_NACC, _D), jnp.float32)]
        + [pltpu.SemaphoreType.DMA] * 18,
    )
    def spmm(h_hbm, src_hbm, dst_hbm, val_hbm, out_hbm,
             sr0, sr1, sr2, sr3, sr4, sr5,
             dr0, dr1, dr2, dr3, dr4, dr5,
             vr0, vr1, vr2, vr3, vr4, vr5,
             rb0, rb1, rb2, rb3, rb4, rb5, acc,
             gs0, gs1, gs2, gs3, gs4, gs5,
             ss0, ss1, ss2, ss3, ss4, ss5,
             is0, is1, is2, is3, is4, is5):
        srcr = (sr0, sr1, sr2, sr3, sr4, sr5)
        dstr = (dr0, dr1, dr2, dr3, dr4, dr5)
        valr = (vr0, vr1, vr2, vr3, vr4, vr5)
        bufs = (rb0, rb1, rb2, rb3, rb4, rb5)
        gsems = (gs0, gs1, gs2, gs3, gs4, gs5)
        ssems = (ss0, ss1, ss2, ss3, ss4, ss5)
        isems = (is0, is1, is2, is3, is4, is5)
        c = lax.axis_index("c")
        s = lax.axis_index("s")
        wid = c * _NS + s
        ebase = wid * _EPT

        def _stage_idx(t, sl, sync=False):
            off = ebase + t * _CHUNK
            if sync:
                pltpu.sync_copy(src_hbm.at[pl.ds(off, _CHUNK)], srcr[sl])
                pltpu.sync_copy(dst_hbm.at[pl.ds(off, _CHUNK)], dstr[sl])
                pltpu.sync_copy(val_hbm.at[pl.ds(off, _CHUNK)], valr[sl])
            else:
                pltpu.async_copy(src_hbm.at[pl.ds(off, _CHUNK)], srcr[sl],
                                 isems[sl])
                pltpu.async_copy(dst_hbm.at[pl.ds(off, _CHUNK)], dstr[sl],
                                 isems[sl])
                pltpu.async_copy(val_hbm.at[pl.ds(off, _CHUNK)], valr[sl],
                                 isems[sl])

        def _wait_idx_sem(t, sl):
            off = ebase + t * _CHUNK
            pltpu.make_async_copy(src_hbm.at[pl.ds(off, _CHUNK)], srcr[sl],
                                  isems[sl]).wait()
            pltpu.make_async_copy(dst_hbm.at[pl.ds(off, _CHUNK)], dstr[sl],
                                  isems[sl]).wait()
            pltpu.make_async_copy(val_hbm.at[pl.ds(off, _CHUNK)], valr[sl],
                                  isems[sl]).wait()

        @pl.loop(0, _CHUNK)
        def _zero_rows(r):
            for d in range(_D // 16):
                rb5[r, pl.ds(d * 16, 16)] = jnp.zeros((16,), jnp.float32)

        _stage_idx(0, 0, sync=True)
        pltpu.async_copy(h_hbm.at[srcr[0]], rb0, gs0)
        _stage_idx(1, 1, sync=True)
        pltpu.async_copy(h_hbm.at[srcr[1]], rb1, gs1)
        _stage_idx(2, 2, sync=True)

        row_base = s * _RPT

        @pl.loop(0, _RPT // 48)
        def _zero_acc(j):
            pltpu.sync_copy(rb5.at[pl.ds(0, 48)],
                            acc.at[pl.ds(row_base + j * 48, 48)])

        @pl.when(s == _NS - 1)
        def _zero_acc_tail():
            pltpu.sync_copy(rb5.at[pl.ds(0, 16)],
                            acc.at[pl.ds(_NS * _RPT, 16)])

        @pl.when(s == 0)
        def _zero_out_pad():
            @pl.loop(0, 5)
            def _zp(j):
                pltpu.sync_copy(
                    rb5.at[pl.ds(0, 48)],
                    out_hbm.at[pl.ds(c * _NPAD + _NACC + j * 48, 48)])

        plsc.subcore_barrier()

        def _scale_rows(buf, vref):
            @pl.loop(0, _CHUNK // 16)
            def _scale(g):
                vb = vref[pl.ds(g * 16, 16)]
                for e in range(16):
                    sc = _bcast_lane(vb, e)
                    r = g * 16 + e
                    for d in range(_D // 16):
                        buf[r, pl.ds(d * 16, 16)] = (
                            buf[r, pl.ds(d * 16, 16)] * sc)

        @pl.loop(0, _NITER)
        def _body(i):
            for j in range(6):
                t = i * 6 + j
                k = j
                k3 = (j + 3) % 6
                k2 = (j + 2) % 6

                def _wait_sca(k3=k3):
                    pltpu.make_async_copy(
                        bufs[k3], acc.at[dstr[k3]], ssems[k3]).wait()
                if j >= 3:
                    _wait_sca()
                else:
                    pl.when(i > 0)(_wait_sca)

                def _stage(t=t, k3=k3):
                    _stage_idx(t + 3, k3)
                if j < 3:
                    _stage()
                else:
                    pl.when(i < _NITER - 1)(_stage)

                def _wait_idx(t=t, k2=k2):
                    _wait_idx_sem(t + 2, k2)
                if j == 0:
                    pl.when(i > 0)(_wait_idx)
                elif j >= 4:
                    pl.when(i < _NITER - 1)(_wait_idx)
                else:
                    _wait_idx()

                def _gather(k2=k2):
                    pltpu.async_copy(h_hbm.at[srcr[k2]], bufs[k2],
                                     gsems[k2])
                if j >= 4:
                    pl.when(i < _NITER - 1)(_gather)
                else:
                    _gather()

                pltpu.make_async_copy(
                    h_hbm.at[srcr[k]], bufs[k], gsems[k]).wait()
                _scale_rows(bufs[k], valr[k])
                pltpu.async_copy(bufs[k], acc.at[dstr[k]], ssems[k],
                                 add=True)

        for tt in (_NCH - 3, _NCH - 2, _NCH - 1):
            pltpu.make_async_copy(
                bufs[tt % 6], acc.at[dstr[tt % 6]], ssems[tt % 6]).wait()

        plsc.subcore_barrier()

        @pl.loop(0, _RPT // 208)
        def _copy_out(j):
            r0 = row_base + j * 208
            pltpu.sync_copy(acc.at[pl.ds(r0, 208)],
                            out_hbm.at[pl.ds(c * _NPAD + r0, 208)])

        @pl.when(s == _NS - 1)
        def _copy_out_tail():
            pltpu.sync_copy(acc.at[pl.ds(_NS * _RPT, 16)],
                            out_hbm.at[pl.ds(c * _NPAD + _NS * _RPT, 16)])

    return spmm(h, src, dst, val)


def _dense_relu(parts, W, b):
    def body(p_ref, w_ref, b_ref, o_ref):
        ssum = p_ref[0] + p_ref[1]
        o_ref[...] = jax.nn.relu(
            jnp.dot(ssum, w_ref[...], preferred_element_type=jnp.float32)
            + b_ref[...])

    return pl.pallas_call(
        body,
        grid=(_NGRID,),
        in_specs=[
            pl.BlockSpec((2, _NBLK, _D), lambda i: (0, i, 0)),
            pl.BlockSpec((_D, _D), lambda i: (0, 0)),
            pl.BlockSpec((1, _D), lambda i: (0, 0)),
        ],
        out_specs=pl.BlockSpec((_NBLK, _D), lambda i: (i, 0)),
        out_shape=jax.ShapeDtypeStruct((_NPAD, _D), jnp.float32),
    )(parts, W, b.reshape(1, _D))


def _head(parts, W, b, batch3, W_p0, b_p0, W_p1, b_p1):
    def body(p_ref, w_ref, b_ref, bat_ref, wp0_ref, bp0_ref, wp1_ref,
             bp1_ref, o_ref, fp_ref):
        i = pl.program_id(0)
        h2 = jax.nn.relu(
            jnp.dot(p_ref[0] + p_ref[1], w_ref[...],
                    preferred_element_type=jnp.float32) + b_ref[...])
        bb = bat_ref[0, 0, :]
        oh = (bb[:, None] == lax.broadcasted_iota(jnp.int32, (_NBLK, _G), 1)
              ).astype(jnp.float32)
        part = lax.dot_general(oh, h2, (((0,), (0,)), ((), ())),
                               preferred_element_type=jnp.float32)

        @pl.when(i == 0)
        def _init():
            fp_ref[...] = part

        @pl.when(i > 0)
        def _accum():
            fp_ref[...] = fp_ref[...] + part

        @pl.when(i == _NGRID - 1)
        def _final():
            z = jax.nn.relu(
                jnp.dot(fp_ref[...], wp0_ref[...],
                        preferred_element_type=jnp.float32) + bp0_ref[...])
            o_ref[...] = (jnp.dot(z, wp1_ref[...],
                                  preferred_element_type=jnp.float32)
                          + bp1_ref[...])

    return pl.pallas_call(
        body,
        grid=(_NGRID,),
        in_specs=[
            pl.BlockSpec((2, _NBLK, _D), lambda i: (0, i, 0)),
            pl.BlockSpec((_D, _D), lambda i: (0, 0)),
            pl.BlockSpec((1, _D), lambda i: (0, 0)),
            pl.BlockSpec((1, 1, _NBLK), lambda i: (i, 0, 0)),
            pl.BlockSpec((_D, _H), lambda i: (0, 0)),
            pl.BlockSpec((1, _H), lambda i: (0, 0)),
            pl.BlockSpec((_H, 1), lambda i: (0, 0)),
            pl.BlockSpec((1, 1), lambda i: (0, 0)),
        ],
        out_specs=pl.BlockSpec((_G, 1), lambda i: (0, 0)),
        out_shape=jax.ShapeDtypeStruct((_G, 1), jnp.float32),
        scratch_shapes=[pltpu.VMEM((_G, _D), jnp.float32)],
    )(parts, W, b.reshape(1, _D), batch3, W_p0, b_p0.reshape(1, _H),
      W_p1, b_p1.reshape(1, 1))


def kernel(node_attr, adj_index, adj_value, batch,
           W_g0, b_g0, W_g1, b_g1, W_p0, b_p0, W_p1, b_p1):
    dst = adj_index[0]
    src = adj_index[1]
    pad_e = _EPAD - _E
    pad_idx = (jnp.arange(pad_e, dtype=jnp.int32) * 61) % _N
    src_p = jnp.concatenate([src.astype(jnp.int32), pad_idx])
    dst_p = jnp.concatenate([dst.astype(jnp.int32), pad_idx])
    val_p = jnp.concatenate([adj_value, jnp.zeros((pad_e,), jnp.float32)])
    batch3 = jnp.concatenate(
        [batch.astype(jnp.int32), jnp.full((_NPAD - _N,), _G, jnp.int32)]
    ).reshape(_NGRID, 1, _NBLK)

    parts0 = _spmm_sc(node_attr, src_p, dst_p, val_p).reshape(_NC, _NPAD, _D)
    h1 = _dense_relu(parts0, W_g0, b_g0)
    parts1 = _spmm_sc(h1, src_p, dst_p, val_p).reshape(_NC, _NPAD, _D)
    return _head(parts1, W_g1, b_g1, batch3, W_p0, b_p0, W_p1, b_p1)
